# baseline (speedup 1.0000x reference)
import functools

import jax
import jax.numpy as jnp
from jax import lax
from jax.experimental import pallas as pl
from jax.experimental.pallas import tpu as pltpu
from jax.experimental.pallas import tpu_sc as plsc

_LIMIT = 8
_THRESHOLD = 0.9
_EPS = 1e-5
_SPAD = 640
_HALO = 144


def _band_kernel(x_ref, out_ref):
    b, s_len, _ = x_ref.shape
    cols = [None] * (_LIMIT * b)
    for bi in range(b):
        x = x_ref[bi]
        for d in range(_LIMIT):
            sh = x if d == 0 else pltpu.roll(x, s_len - d, axis=0)
            c = jnp.sum(x * sh, axis=1, keepdims=True)
            if d > 0:
                sub = jax.lax.broadcasted_iota(jnp.int32, (s_len, 1), 0)
                c = jnp.where(sub < s_len - d, c, 0.0)
            cols[d * b + bi] = c
    m = jnp.concatenate(cols, axis=1)
    mt = m.T
    out_ref[...] = jnp.concatenate(
        [mt, jnp.zeros((_LIMIT * b, _SPAD - s_len), jnp.float32)], axis=1)


def _rsqrt(v):
    i = lax.bitcast_convert_type(v, jnp.int32)
    i = 0x5F3759DF - lax.shift_right_arithmetic(i, 1)
    y = lax.bitcast_convert_type(i, jnp.float32)
    for _ in range(3):
        y = y * (1.5 - 0.5 * v * y * y)
    return y


def _sqrt(v):
    v = jnp.maximum(v, 0.0)
    return v * _rsqrt(jnp.maximum(v, 1e-30))


def _sc_window_kernel(band_hbm, rm_hbm, worst_hbm, incl_hbm,
                      a_v, rm_v, w_v, i_v):
    wid = lax.axis_index("s") * 2 + lax.axis_index("c")
    b_idx = wid // 4
    t0 = (wid % 4) * 128

    for d in range(_LIMIT):
        pltpu.sync_copy(
            band_hbm.at[pl.ds((d * 8 + b_idx) * _SPAD + t0, _HALO)],
            a_v.at[pl.ds(d * _HALO, _HALO)])
    pltpu.sync_copy(rm_hbm.at[pl.ds(b_idx * _SPAD + t0, _HALO)], rm_v)

    for i in range(8):
        s0 = i * 16

        def ld(d, c):
            return a_v[pl.ds(d * _HALO + s0 + c, 16)]

        n0 = ld(0, 0)
        gns = [None] * _LIMIT
        for j in range(_LIMIT):
            nj = n0 if j == 0 else ld(0, j)
            gns[j] = jnp.maximum(_sqrt(nj), _EPS)

        rows = [n0] + [None] * (_LIMIT - 1)
        swin = n0
        w_v[pl.ds(s0, 16)] = n0 / (jnp.maximum(_sqrt(n0), _EPS) * gns[0])
        i_v[pl.ds(s0, 16)] = jnp.full((16,), 1, jnp.int32)

        regw = rm_v[pl.ds(s0, 16)]
        for L in range(2, _LIMIT + 1):
            for j in range(L - 1):
                rows[j] = rows[j] + ld(L - 1 - j, j)
            new = ld(L - 1, 0)
            for k in range(1, L):
                new = new + ld(L - 1 - k, k)
            swin = swin + 2.0 * new - ld(0, L - 1)
            rows[L - 1] = new
            regw = regw * rm_v[pl.ds(s0 + L - 1, 16)]

            cn = jnp.maximum(_sqrt(swin), L * _EPS)
            worst = None
            for j in range(L):
                s_j = rows[j] / (cn * gns[j])
                worst = s_j if worst is None else jnp.minimum(worst, s_j)
            w_v[pl.ds((L - 1) * 128 + s0, 16)] = worst
            i_v[pl.ds((L - 1) * 128 + s0, 16)] = jnp.where(
                (worst >= _THRESHOLD) & (regw == 1),
                jnp.full((16,), 1, jnp.int32), jnp.full((16,), 0, jnp.int32))

    for L in range(_LIMIT):
        off = (b_idx * _LIMIT + L) * 512 + t0
        pltpu.sync_copy(w_v.at[pl.ds(L * 128, 128)],
                        worst_hbm.at[pl.ds(off, 128)])
        pltpu.sync_copy(i_v.at[pl.ds(L * 128, 128)],
                        incl_hbm.at[pl.ds(off, 128)])


def kernel(batch_sequence_tensors, regular_tokens_mask):
    x = batch_sequence_tensors
    rm = regular_tokens_mask.astype(jnp.int32)
    b, s_len, d_len = x.shape

    band = pl.pallas_call(
        _band_kernel,
        out_shape=jax.ShapeDtypeStruct((_LIMIT * b, _SPAD), jnp.float32),
    )(x)

    rmp = jnp.pad(rm, ((0, 0), (0, _SPAD - s_len))).reshape(-1)

    sc_fn = functools.partial(
        pl.kernel,
        mesh=plsc.VectorSubcoreMesh(core_axis_name="c", subcore_axis_name="s"),
        out_type=(
            jax.ShapeDtypeStruct((b * _LIMIT * s_len,), jnp.float32),
            jax.ShapeDtypeStruct((b * _LIMIT * s_len,), jnp.int32),
        ),
        scratch_types=[
            pltpu.VMEM((_LIMIT * _HALO,), jnp.float32),
            pltpu.VMEM((_HALO,), jnp.int32),
            pltpu.VMEM((_LIMIT * 128,), jnp.float32),
            pltpu.VMEM((_LIMIT * 128,), jnp.int32),
        ],
    )(_sc_window_kernel)
    worst_f, incl_f = sc_fn(band.reshape(-1), rmp)
    worst8 = worst_f.reshape(b, _LIMIT, s_len)
    incl8 = incl_f.reshape(b, _LIMIT, s_len)

    worst_all = jnp.concatenate(
        [worst8[:, L - 1, : s_len - L + 1] for L in range(1, _LIMIT + 1)],
        axis=1)
    include = jnp.concatenate(
        [incl8[:, L - 1, : s_len - L + 1] != 0 for L in range(1, _LIMIT + 1)],
        axis=1)
    return worst_all, include


# speedup vs baseline: 1.1322x; 1.1322x over previous
import functools

import jax
import jax.numpy as jnp
from jax import lax
from jax.experimental import pallas as pl
from jax.experimental.pallas import tpu as pltpu
from jax.experimental.pallas import tpu_sc as plsc

_LIMIT = 8
_THRESHOLD = 0.9
_EPS = 1e-5
_SPAD = 640
_HALO = 144


def _band_kernel(x_ref, out_ref):
    b, s_len, _ = x_ref.shape
    cols = [None] * (_LIMIT * b)
    for bi in range(b):
        x = x_ref[bi]
        for d in range(_LIMIT):
            sh = x if d == 0 else pltpu.roll(x, s_len - d, axis=0)
            c = jnp.sum(x * sh, axis=1, keepdims=True)
            if d > 0:
                sub = jax.lax.broadcasted_iota(jnp.int32, (s_len, 1), 0)
                c = jnp.where(sub < s_len - d, c, 0.0)
            cols[d * b + bi] = c
    m = jnp.concatenate(cols, axis=1)
    mt = m.T
    out_ref[...] = jnp.concatenate(
        [mt, jnp.zeros((_LIMIT * b, _SPAD - s_len), jnp.float32)], axis=1)


def _rsqrt(v):
    i = lax.bitcast_convert_type(v, jnp.int32)
    i = 0x5F3759DF - lax.shift_right_arithmetic(i, 1)
    y = lax.bitcast_convert_type(i, jnp.float32)
    for _ in range(3):
        y = y * (1.5 - 0.5 * v * y * y)
    return y


def _sqrt(v):
    v = jnp.maximum(v, 0.0)
    return v * _rsqrt(jnp.maximum(v, 1e-30))


def _sc_window_kernel(band_hbm, rm_hbm, worst_hbm, incl_hbm,
                      a_v, rm_v, w_v, i_v, sn_v, sem):
    wid = lax.axis_index("s") * 2 + lax.axis_index("c")
    b_idx = wid // 4
    t0 = (wid % 4) * 128

    cps = [pltpu.async_copy(
        band_hbm.at[pl.ds((d * 8 + b_idx) * _SPAD + t0, _HALO)],
        a_v.at[pl.ds(d * _HALO, _HALO)], sem) for d in range(_LIMIT)]
    cps.append(pltpu.async_copy(
        rm_hbm.at[pl.ds(b_idx * _SPAD + t0, _HALO)], rm_v, sem))
    for c in cps:
        c.wait()

    for i in range(_HALO // 16):
        sn_v[pl.ds(i * 16, 16)] = jnp.maximum(
            _sqrt(a_v[pl.ds(i * 16, 16)]), _EPS)

    for i in range(8):
        s0 = i * 16

        def ld(d, c):
            return a_v[pl.ds(d * _HALO + s0 + c, 16)]

        n0 = ld(0, 0)
        gns = [sn_v[pl.ds(s0 + j, 16)] for j in range(_LIMIT)]

        rows = [n0] + [None] * (_LIMIT - 1)
        swin = n0
        w_v[pl.ds(s0, 16)] = n0 / (gns[0] * gns[0])
        i_v[pl.ds(s0, 16)] = jnp.full((16,), 1, jnp.int32)

        regw = rm_v[pl.ds(s0, 16)]
        for L in range(2, _LIMIT + 1):
            for j in range(L - 1):
                rows[j] = rows[j] + ld(L - 1 - j, j)
            new = ld(L - 1, 0)
            for k in range(1, L):
                new = new + ld(L - 1 - k, k)
            swin = swin + 2.0 * new - ld(0, L - 1)
            rows[L - 1] = new
            regw = regw * rm_v[pl.ds(s0 + L - 1, 16)]

            cn = jnp.maximum(_sqrt(swin), L * _EPS)
            worst = None
            for j in range(L):
                s_j = rows[j] / (cn * gns[j])
                worst = s_j if worst is None else jnp.minimum(worst, s_j)
            w_v[pl.ds((L - 1) * 128 + s0, 16)] = worst
            i_v[pl.ds((L - 1) * 128 + s0, 16)] = jnp.where(
                (worst >= _THRESHOLD) & (regw == 1),
                jnp.full((16,), 1, jnp.int32), jnp.full((16,), 0, jnp.int32))

    outs = []
    for L in range(_LIMIT):
        off = (b_idx * _LIMIT + L) * 512 + t0
        outs.append(pltpu.async_copy(w_v.at[pl.ds(L * 128, 128)],
                                     worst_hbm.at[pl.ds(off, 128)], sem))
        outs.append(pltpu.async_copy(i_v.at[pl.ds(L * 128, 128)],
                                     incl_hbm.at[pl.ds(off, 128)], sem))
    for c in outs:
        c.wait()


def kernel(batch_sequence_tensors, regular_tokens_mask):
    x = batch_sequence_tensors
    rm = regular_tokens_mask.astype(jnp.int32)
    b, s_len, d_len = x.shape

    band = pl.pallas_call(
        _band_kernel,
        out_shape=jax.ShapeDtypeStruct((_LIMIT * b, _SPAD), jnp.float32),
    )(x)

    rmp = jnp.pad(rm, ((0, 0), (0, _SPAD - s_len))).reshape(-1)

    sc_fn = functools.partial(
        pl.kernel,
        mesh=plsc.VectorSubcoreMesh(core_axis_name="c", subcore_axis_name="s"),
        out_type=(
            jax.ShapeDtypeStruct((b * _LIMIT * s_len,), jnp.float32),
            jax.ShapeDtypeStruct((b * _LIMIT * s_len,), jnp.int32),
        ),
        scratch_types=[
            pltpu.VMEM((_LIMIT * _HALO,), jnp.float32),
            pltpu.VMEM((_HALO,), jnp.int32),
            pltpu.VMEM((_LIMIT * 128,), jnp.float32),
            pltpu.VMEM((_LIMIT * 128,), jnp.int32),
            pltpu.VMEM((_HALO,), jnp.float32),
            pltpu.SemaphoreType.DMA,
        ],
    )(_sc_window_kernel)
    worst_f, incl_f = sc_fn(band.reshape(-1), rmp)
    worst8 = worst_f.reshape(b, _LIMIT, s_len)
    incl8 = incl_f.reshape(b, _LIMIT, s_len)

    worst_all = jnp.concatenate(
        [worst8[:, L - 1, : s_len - L + 1] for L in range(1, _LIMIT + 1)],
        axis=1)
    include = jnp.concatenate(
        [incl8[:, L - 1, : s_len - L + 1] != 0 for L in range(1, _LIMIT + 1)],
        axis=1)
    return worst_all, include


# speedup vs baseline: 1.1796x; 1.0418x over previous
import functools

import jax
import jax.numpy as jnp
from jax import lax
from jax.experimental import pallas as pl
from jax.experimental.pallas import tpu as pltpu
from jax.experimental.pallas import tpu_sc as plsc

_LIMIT = 8
_THRESHOLD = 0.9
_EPS = 1e-5
_SPAD = 640
_HALO = 256
_SNW = 144


def _band_kernel(x_ref, out_ref):
    b, s_len, _ = x_ref.shape
    cols = [None] * (_LIMIT * b)
    for bi in range(b):
        x = x_ref[bi]
        for d in range(_LIMIT):
            sh = x if d == 0 else pltpu.roll(x, s_len - d, axis=0)
            c = jnp.sum(x * sh, axis=1, keepdims=True)
            if d > 0:
                sub = jax.lax.broadcasted_iota(jnp.int32, (s_len, 1), 0)
                c = jnp.where(sub < s_len - d, c, 0.0)
            cols[bi * _LIMIT + d] = c
    m = jnp.concatenate(cols, axis=1)
    mt = m.T
    out_ref[...] = jnp.concatenate(
        [mt, jnp.zeros((_LIMIT * b, _SPAD - s_len), jnp.float32)], axis=1)


def _rsqrt(v):
    i = lax.bitcast_convert_type(v, jnp.int32)
    i = 0x5F3759DF - lax.shift_right_arithmetic(i, 1)
    y = lax.bitcast_convert_type(i, jnp.float32)
    for _ in range(3):
        y = y * (1.5 - 0.5 * v * y * y)
    return y


def _sqrt(v):
    v = jnp.maximum(v, 0.0)
    return v * _rsqrt(jnp.maximum(v, 1e-30))


def _sc_window_kernel(band_hbm, rm_hbm, worst_hbm, incl_hbm,
                      a_v, rm_v, w_v, i_v, sn_v, sem):
    wid = lax.axis_index("s") * 2 + lax.axis_index("c")
    b_idx = wid // 4
    t0 = (wid % 4) * 128

    cps = [pltpu.async_copy(
        band_hbm.at[pl.ds(b_idx * _LIMIT, _LIMIT), pl.ds(t0, _HALO)],
        a_v, sem)]
    cps.append(pltpu.async_copy(
        rm_hbm.at[pl.ds(b_idx * _SPAD + t0, _HALO)], rm_v, sem))
    for c in cps:
        c.wait()

    for i in range(_SNW // 16):
        sn_v[pl.ds(i * 16, 16)] = jnp.maximum(
            _sqrt(a_v[0, pl.ds(i * 16, 16)]), _EPS)

    for i in range(8):
        s0 = i * 16

        def ld(d, c):
            return a_v[d, pl.ds(s0 + c, 16)]

        n0 = ld(0, 0)
        gns = [sn_v[pl.ds(s0 + j, 16)] for j in range(_LIMIT)]

        rows = [n0] + [None] * (_LIMIT - 1)
        swin = n0
        w_v[pl.ds(s0, 16)] = n0 / (gns[0] * gns[0])
        i_v[pl.ds(s0, 16)] = jnp.full((16,), 1, jnp.int32)

        regw = rm_v[pl.ds(s0, 16)]
        for L in range(2, _LIMIT + 1):
            for j in range(L - 1):
                rows[j] = rows[j] + ld(L - 1 - j, j)
            new = ld(L - 1, 0)
            for k in range(1, L):
                new = new + ld(L - 1 - k, k)
            swin = swin + 2.0 * new - ld(0, L - 1)
            rows[L - 1] = new
            regw = regw * rm_v[pl.ds(s0 + L - 1, 16)]

            cn = jnp.maximum(_sqrt(swin), L * _EPS)
            worst = None
            for j in range(L):
                s_j = rows[j] / (cn * gns[j])
                worst = s_j if worst is None else jnp.minimum(worst, s_j)
            w_v[pl.ds((L - 1) * 128 + s0, 16)] = worst
            i_v[pl.ds((L - 1) * 128 + s0, 16)] = jnp.where(
                (worst >= _THRESHOLD) & (regw == 1),
                jnp.full((16,), 1, jnp.int32), jnp.full((16,), 0, jnp.int32))

    outs = []
    for L in range(_LIMIT):
        off = (b_idx * _LIMIT + L) * 512 + t0
        outs.append(pltpu.async_copy(w_v.at[pl.ds(L * 128, 128)],
                                     worst_hbm.at[pl.ds(off, 128)], sem))
        outs.append(pltpu.async_copy(i_v.at[pl.ds(L * 128, 128)],
                                     incl_hbm.at[pl.ds(off, 128)], sem))
    for c in outs:
        c.wait()


def _assemble_kernel(wf_ref, if_ref, worst_ref, incl_ref):
    b = worst_ref.shape[0]
    for bi in range(b):
        off = 0
        for li in range(_LIMIT):
            w = 512 - li
            row_w = wf_ref[pl.ds((bi * _LIMIT + li) * 512, 512)]
            row_i = if_ref[pl.ds((bi * _LIMIT + li) * 512, 512)]
            worst_ref[bi, off:off + w] = row_w[:w]
            incl_ref[bi, off:off + w] = row_i[:w]
            off += w


def kernel(batch_sequence_tensors, regular_tokens_mask):
    x = batch_sequence_tensors
    rm = regular_tokens_mask.astype(jnp.int32)
    b, s_len, d_len = x.shape

    band = pl.pallas_call(
        _band_kernel,
        out_shape=jax.ShapeDtypeStruct((_LIMIT * b, _SPAD), jnp.float32),
    )(x)

    rmp = jnp.pad(rm, ((0, 0), (0, _SPAD - s_len))).reshape(-1)

    sc_fn = functools.partial(
        pl.kernel,
        mesh=plsc.VectorSubcoreMesh(core_axis_name="c", subcore_axis_name="s"),
        out_type=(
            jax.ShapeDtypeStruct((b * _LIMIT * s_len,), jnp.float32),
            jax.ShapeDtypeStruct((b * _LIMIT * s_len,), jnp.int32),
        ),
        scratch_types=[
            pltpu.VMEM((_LIMIT, _HALO), jnp.float32),
            pltpu.VMEM((_HALO,), jnp.int32),
            pltpu.VMEM((_LIMIT * 128,), jnp.float32),
            pltpu.VMEM((_LIMIT * 128,), jnp.int32),
            pltpu.VMEM((_HALO,), jnp.float32),
            pltpu.SemaphoreType.DMA,
        ],
    )(_sc_window_kernel)
    worst_f, incl_f = sc_fn(band, rmp)

    n_out = _LIMIT * s_len - (_LIMIT * (_LIMIT - 1)) // 2
    worst_all, incl = pl.pallas_call(
        _assemble_kernel,
        out_shape=(
            jax.ShapeDtypeStruct((b, n_out), jnp.float32),
            jax.ShapeDtypeStruct((b, n_out), jnp.int32),
        ),
    )(worst_f, incl_f)
    return worst_all, incl != 0
